# Initial kernel scaffold; baseline (speedup 1.0000x reference)
import jax
import jax.numpy as jnp
from jax.experimental import pallas as pl


def kernel(t_embed, v_embed, a_embed, wv, wt, wa_t, w1, w2, wa, wa_v, ptr_t, a_list_t, v_list_t, ptr_v, a_list_v, t_list_v):
    raise NotImplementedError("write your pallas kernel here")


# speedup vs baseline: 23.2527x; 23.2527x over previous
import functools

import jax
import jax.numpy as jnp
from jax import lax
from jax.experimental import pallas as pl
from jax.experimental.pallas import tpu as pltpu
from jax.experimental.pallas import tpu_sc as plsc

NT = 10000
NV = 10000
NA = 10000
E = 160000
D = 256
H = 128
NC = 2
NS = 16
L = 16
CH = 128
NCHUNK = E // CH
PTR_LEN = 10016
NSEG_HALF = 5000
SEG_PT = 312
R_SP = 5016
BSEARCH_ITERS = 9

BN = 1000


def _mm_kernel(x_ref, w_ref, o_ref):
    o_ref[...] = jnp.dot(x_ref[...], w_ref[...],
                         preferred_element_type=jnp.float32)


def _matmul(x, w):
    n, k = x.shape
    m = w.shape[1]
    return pl.pallas_call(
        _mm_kernel,
        grid=(n // BN,),
        in_specs=[pl.BlockSpec((BN, k), lambda i: (i, 0)),
                  pl.BlockSpec((k, m), lambda i: (0, 0))],
        out_specs=pl.BlockSpec((BN, m), lambda i: (i, 0)),
        out_shape=jax.ShapeDtypeStruct((n, m), jnp.float32),
    )(x, w)


def _mm_split_kernel(x_ref, w_ref, o_ref):
    o_ref[0] = jnp.dot(x_ref[...], w_ref[...],
                       preferred_element_type=jnp.float32)


def _matmul_split(x, w):
    n = x.shape[0]
    out = pl.pallas_call(
        _mm_split_kernel,
        grid=(NC, n // BN),
        in_specs=[pl.BlockSpec((BN, D), lambda c, i: (i, 0)),
                  pl.BlockSpec((D, H), lambda c, i: (0, c))],
        out_specs=pl.BlockSpec((1, BN, H), lambda c, i: (c, i, 0)),
        out_shape=jax.ShapeDtypeStruct((NC, n, H), jnp.float32),
    )(x, w)
    return out.reshape(NC * n, H)


def _mm2_kernel(x1_ref, w1_ref, x2_ref, w2_ref, o_ref):
    o_ref[...] = (
        jnp.dot(x1_ref[...], w1_ref[...], preferred_element_type=jnp.float32)
        + jnp.dot(x2_ref[...], w2_ref[...],
                  preferred_element_type=jnp.float32))


def _matmul2(x1, w1_, x2, w2_):
    n, k1 = x1.shape
    k2 = x2.shape[1]
    m = w1_.shape[1]
    return pl.pallas_call(
        _mm2_kernel,
        grid=(n // BN,),
        in_specs=[pl.BlockSpec((BN, k1), lambda i: (i, 0)),
                  pl.BlockSpec((k1, m), lambda i: (0, 0)),
                  pl.BlockSpec((BN, k2), lambda i: (i, 0)),
                  pl.BlockSpec((k2, m), lambda i: (0, 0))],
        out_specs=pl.BlockSpec((BN, m), lambda i: (i, 0)),
        out_shape=jax.ShapeDtypeStruct((n, m), jnp.float32),
    )(x1, w1_, x2, w2_)


def _sc_body(tab_vt, tab_at, vlist_t, alist_t, ptr_t,
             tab_tv, tab_av, tlist_v, alist_v, ptr_v,
             out_t, out_v,
             ptrbuf, iXbuf, iYbuf, segbuf, xbuf, ybuf, zbuf, acc,
             sem1, sem2):
    c = lax.axis_index("c")
    s = lax.axis_index("s")
    zeros = jnp.zeros((L,), jnp.float32)
    last = s == NS - 1

    def zrow(r, carry):
        for g in range(H // L):
            zbuf[r, pl.ds(g * L, L)] = zeros
        return carry
    lax.fori_loop(0, CH, zrow, 0)

    def run_pass(tab_x, tab_y, xlist, ylist, seg_base, nrow_x, nrow_y,
                 out_ref):
        seg_lo = seg_base + s * SEG_PT
        seg_hi = seg_base + jnp.where(last, NSEG_HALF, (s + 1) * SEG_PT)
        trash = NSEG_HALF + s
        tail = NSEG_HALF - NS * SEG_PT

        for off in range(0, SEG_PT, CH):
            step = min(CH, SEG_PT - off)
            pltpu.sync_copy(zbuf.at[pl.ds(0, step)],
                            acc.at[pl.ds(s * SEG_PT + off, step)])

        @pl.when(last)
        def _():
            pltpu.sync_copy(zbuf.at[pl.ds(0, tail)],
                            acc.at[pl.ds(NS * SEG_PT, tail)])

        p_lo = plsc.load_gather(
            ptrbuf, [jnp.broadcast_to(seg_lo, (L,)).astype(jnp.int32)])
        p_hi = plsc.load_gather(
            ptrbuf, [jnp.broadcast_to(seg_hi, (L,)).astype(jnp.int32)])
        k0 = p_lo[0] // CH
        k1 = (p_hi[0] + (CH - 1)) // CH
        lovec = jnp.broadcast_to(seg_lo, (L,)).astype(jnp.int32)
        hivec = jnp.broadcast_to(seg_hi - 1, (L,)).astype(jnp.int32)
        off_x = c * nrow_x
        off_y = c * nrow_y

        def chunk_body(k, carry):
            base = k * CH
            pltpu.sync_copy(xlist.at[pl.ds(base, CH)], iXbuf)
            pltpu.sync_copy(ylist.at[pl.ds(base, CH)], iYbuf)
            for g in range(CH // L):
                sl = pl.ds(g * L, L)
                iXbuf[sl] = iXbuf[sl] + off_x
                iYbuf[sl] = iYbuf[sl] + off_y
                e = (jnp.full((L,), base + g * L, jnp.int32)
                     + lax.iota(jnp.int32, L))
                own = (e >= p_lo) & (e < p_hi)
                lo = lovec
                hi = hivec
                for _ in range(BSEARCH_ITERS):
                    mid = (lo + hi + 1) >> 1
                    pm = plsc.load_gather(ptrbuf, [mid])
                    gele = pm <= e
                    lo = jnp.where(gele, mid, lo)
                    hi = jnp.where(gele, hi, mid - 1)
                segbuf[sl] = jnp.where(own, lo - seg_base, trash)
            cp1 = pltpu.async_copy(tab_x.at[iXbuf], xbuf, sem1)
            cp2 = pltpu.async_copy(tab_y.at[iYbuf], ybuf, sem2)
            cp1.wait()
            cp2.wait()

            def mrow(r, mcarry):
                for g in range(H // L):
                    sl2 = pl.ds(g * L, L)
                    xbuf[r, sl2] = xbuf[r, sl2] * ybuf[r, sl2]
                return mcarry
            lax.fori_loop(0, CH, mrow, 0)
            pltpu.sync_copy(xbuf, acc.at[segbuf], add=True)
            return carry

        lax.fori_loop(k0, k1, chunk_body, 0)

        pltpu.sync_copy(acc.at[pl.ds(s * SEG_PT, SEG_PT)],
                        out_ref.at[c].at[pl.ds(seg_base + s * SEG_PT,
                                               SEG_PT)])

        @pl.when(last)
        def _():
            pltpu.sync_copy(acc.at[pl.ds(NS * SEG_PT, tail)],
                            out_ref.at[c].at[pl.ds(seg_base + NS * SEG_PT,
                                                   tail)])
        plsc.subcore_barrier()

    def run_side(tab_x, tab_y, xlist, ylist, ptr_hbm, nrow_x, nrow_y,
                 out_ref):
        pltpu.sync_copy(ptr_hbm, ptrbuf)
        for seg_base in range(0, NT, NSEG_HALF):
            run_pass(tab_x, tab_y, xlist, ylist, seg_base,
                     nrow_x, nrow_y, out_ref)

    run_side(tab_vt, tab_at, vlist_t, alist_t, ptr_t, NV, NA, out_t)
    run_side(tab_tv, tab_av, tlist_v, alist_v, ptr_v, NT, NA, out_v)


_sc_segsum = functools.partial(
    pl.kernel,
    out_type=[jax.ShapeDtypeStruct((NC, NT, H), jnp.float32),
              jax.ShapeDtypeStruct((NC, NV, H), jnp.float32)],
    mesh=plsc.VectorSubcoreMesh(core_axis_name="c", subcore_axis_name="s"),
    compiler_params=pltpu.CompilerParams(needs_layout_passes=False),
    scratch_types=[
        pltpu.VMEM((PTR_LEN,), jnp.int32),
        pltpu.VMEM((CH,), jnp.int32),
        pltpu.VMEM((CH,), jnp.int32),
        pltpu.VMEM((CH,), jnp.int32),
        pltpu.VMEM((CH, H), jnp.float32),
        pltpu.VMEM((CH, H), jnp.float32),
        pltpu.VMEM((CH, H), jnp.float32),
        pltpu.VMEM_SHARED((R_SP, H), jnp.float32),
        pltpu.SemaphoreType.DMA,
        pltpu.SemaphoreType.DMA,
    ],
)(_sc_body)


def kernel(t_embed, v_embed, a_embed, wv, wt, wa_t, w1, w2, wa, wa_v,
           ptr_t, a_list_t, v_list_t, ptr_v, a_list_v, t_list_v):
    f32 = jnp.float32
    i32 = jnp.int32
    t_embed = t_embed.astype(f32)
    v_embed = v_embed.astype(f32)
    a_embed = a_embed.astype(f32)

    tab_vt = _matmul_split(v_embed, wv.T)
    tab_at = _matmul_split(a_embed, wa_v.T)
    tab_tv = _matmul_split(t_embed, wt.T)
    tab_av = _matmul_split(a_embed, wa_t.T)

    vlt = v_list_t.astype(i32)
    alt = a_list_t.astype(i32)
    tlv = t_list_v.astype(i32)
    alv = a_list_v.astype(i32)
    ptrt = jnp.pad(ptr_t.astype(i32), (0, PTR_LEN - (NT + 1)), mode="edge")
    ptrv = jnp.pad(ptr_v.astype(i32), (0, PTR_LEN - (NV + 1)), mode="edge")

    out_t2, out_v2 = _sc_segsum(tab_vt, tab_at, vlt, alt, ptrt,
                                tab_tv, tab_av, tlv, alv, ptrv)
    seg_t = jnp.concatenate([out_t2[0], out_t2[1]], axis=1)
    seg_v = jnp.concatenate([out_v2[0], out_v2[1]], axis=1)

    t_up = _matmul2(t_embed, w1[:, :D].T, seg_t, w1[:, D:].T)
    v_up = _matmul2(v_embed, w2[:, :D].T, seg_v, w2[:, D:].T)
    a_out = _matmul(a_embed, wa)
    return (t_up, v_up, a_out)


# speedup vs baseline: 32.9406x; 1.4166x over previous
import functools

import jax
import jax.numpy as jnp
from jax import lax
from jax.experimental import pallas as pl
from jax.experimental.pallas import tpu as pltpu
from jax.experimental.pallas import tpu_sc as plsc

NT = 10000
NV = 10000
NA = 10000
E = 160000
D = 256
H = 128
NC = 2
NS = 16
L = 16
CH = 128
NCHUNK = E // CH
PTR_LEN = 10016
NSEG_HALF = 5000
SEG_PT = 312
R_SP = 5016
ZROWS = 32
BSEARCH_ITERS = 9

BN = 1000


def _mm_kernel(x_ref, w_ref, o_ref):
    o_ref[...] = jnp.dot(x_ref[...], w_ref[...],
                         preferred_element_type=jnp.float32)


def _matmul(x, w):
    n, k = x.shape
    m = w.shape[1]
    return pl.pallas_call(
        _mm_kernel,
        grid=(n // BN,),
        in_specs=[pl.BlockSpec((BN, k), lambda i: (i, 0)),
                  pl.BlockSpec((k, m), lambda i: (0, 0))],
        out_specs=pl.BlockSpec((BN, m), lambda i: (i, 0)),
        out_shape=jax.ShapeDtypeStruct((n, m), jnp.float32),
    )(x, w)


def _mm_split_kernel(x_ref, w_ref, o_ref):
    o_ref[0] = jnp.dot(x_ref[...], w_ref[...],
                       preferred_element_type=jnp.float32)


def _matmul_split(x, w):
    n = x.shape[0]
    out = pl.pallas_call(
        _mm_split_kernel,
        grid=(NC, n // BN),
        in_specs=[pl.BlockSpec((BN, D), lambda c, i: (i, 0)),
                  pl.BlockSpec((D, H), lambda c, i: (0, c))],
        out_specs=pl.BlockSpec((1, BN, H), lambda c, i: (c, i, 0)),
        out_shape=jax.ShapeDtypeStruct((NC, n, H), jnp.float32),
    )(x, w)
    return out.reshape(NC * n, H)


def _mm2_kernel(x1_ref, w1_ref, x2_ref, w2_ref, o_ref):
    o_ref[...] = (
        jnp.dot(x1_ref[...], w1_ref[...], preferred_element_type=jnp.float32)
        + jnp.dot(x2_ref[...], w2_ref[...],
                  preferred_element_type=jnp.float32))


def _matmul2(x1, w1_, x2, w2_):
    n, k1 = x1.shape
    k2 = x2.shape[1]
    m = w1_.shape[1]
    return pl.pallas_call(
        _mm2_kernel,
        grid=(n // BN,),
        in_specs=[pl.BlockSpec((BN, k1), lambda i: (i, 0)),
                  pl.BlockSpec((k1, m), lambda i: (0, 0)),
                  pl.BlockSpec((BN, k2), lambda i: (i, 0)),
                  pl.BlockSpec((k2, m), lambda i: (0, 0))],
        out_specs=pl.BlockSpec((BN, m), lambda i: (i, 0)),
        out_shape=jax.ShapeDtypeStruct((n, m), jnp.float32),
    )(x1, w1_, x2, w2_)


def _sc_body(tab_vt, tab_at, vlist_t, alist_t, ptr_t,
             tab_tv, tab_av, tlist_v, alist_v, ptr_v,
             out_t, out_v,
             ptrbuf, iXbuf, iYbuf, segbuf, xbuf, ybuf,
             iXbuf2, iYbuf2, segbuf2, xbuf2, ybuf2, zbuf, acc,
             semx0, semy0, semx1, semy1, sems):
    c = lax.axis_index("c")
    s = lax.axis_index("s")
    zeros = jnp.zeros((L,), jnp.float32)
    last = s == NS - 1

    def zrow(r, carry):
        for g in range(H // L):
            zbuf[r, pl.ds(g * L, L)] = zeros
        return carry
    lax.fori_loop(0, ZROWS, zrow, 0)

    def run_pass(tab_x, tab_y, xlist, ylist, seg_base, nrow_x, nrow_y,
                 out_ref):
        seg_lo = seg_base + s * SEG_PT
        seg_hi = seg_base + jnp.where(last, NSEG_HALF, (s + 1) * SEG_PT)
        trash = NSEG_HALF + s
        tail = NSEG_HALF - NS * SEG_PT

        for off in range(0, SEG_PT, ZROWS):
            step = min(ZROWS, SEG_PT - off)
            pltpu.sync_copy(zbuf.at[pl.ds(0, step)],
                            acc.at[pl.ds(s * SEG_PT + off, step)])

        @pl.when(last)
        def _():
            pltpu.sync_copy(zbuf.at[pl.ds(0, tail)],
                            acc.at[pl.ds(NS * SEG_PT, tail)])

        p_lo = plsc.load_gather(
            ptrbuf, [jnp.broadcast_to(seg_lo, (L,)).astype(jnp.int32)])
        p_hi = plsc.load_gather(
            ptrbuf, [jnp.broadcast_to(seg_hi, (L,)).astype(jnp.int32)])
        k0 = p_lo[0] // CH
        k1 = (p_hi[0] + (CH - 1)) // CH
        lovec = jnp.broadcast_to(seg_lo, (L,)).astype(jnp.int32)
        hivec = jnp.broadcast_to(seg_hi - 1, (L,)).astype(jnp.int32)
        off_x = c * nrow_x
        off_y = c * nrow_y

        bufs = ((iXbuf, iYbuf, segbuf, xbuf, ybuf, semx0, semy0),
                (iXbuf2, iYbuf2, segbuf2, xbuf2, ybuf2, semx1, semy1))

        def prefetch(k, bi):
            iXb, iYb, sgb, xb, yb, sx, sy = bufs[bi]
            base = k * CH
            pltpu.sync_copy(xlist.at[pl.ds(base, CH)], iXb)
            pltpu.sync_copy(ylist.at[pl.ds(base, CH)], iYb)
            for g in range(CH // L):
                sl = pl.ds(g * L, L)
                iXb[sl] = iXb[sl] + off_x
                iYb[sl] = iYb[sl] + off_y
                e = (jnp.full((L,), base + g * L, jnp.int32)
                     + lax.iota(jnp.int32, L))
                own = (e >= p_lo) & (e < p_hi)
                lo = lovec
                hi = hivec
                for _ in range(BSEARCH_ITERS):
                    mid = (lo + hi + 1) >> 1
                    pm = plsc.load_gather(ptrbuf, [mid])
                    gele = pm <= e
                    lo = jnp.where(gele, mid, lo)
                    hi = jnp.where(gele, hi, mid - 1)
                sgb[sl] = jnp.where(own, lo - seg_base, trash)
            pltpu.async_copy(tab_x.at[iXb], xb, sx)
            pltpu.async_copy(tab_y.at[iYb], yb, sy)

        def consume(k, bi):
            iXb, iYb, sgb, xb, yb, sx, sy = bufs[bi]
            pltpu.make_async_copy(tab_x.at[iXb], xb, sx).wait()
            pltpu.make_async_copy(tab_y.at[iYb], yb, sy).wait()

            def mrow(r, mcarry):
                for g in range(H // L):
                    sl2 = pl.ds(g * L, L)
                    xb[r, sl2] = xb[r, sl2] * yb[r, sl2]
                return mcarry
            lax.fori_loop(0, CH, mrow, 0)
            pltpu.async_copy(xb, acc.at[sgb], sems, add=True)

        def wait_scatter(bi):
            iXb, iYb, sgb, xb, yb, sx, sy = bufs[bi]
            pltpu.make_async_copy(xb, acc.at[sgb], sems).wait()

        @pl.when(k0 < k1)
        def _():
            prefetch(k0, 0)

        def pair_body(j, carry):
            for p in range(2):
                k = k0 + 2 * j + p

                @pl.when(k < k1)
                def _():
                    @pl.when(k > k0)
                    def _():
                        wait_scatter(1 - p)

                    @pl.when(k + 1 < k1)
                    def _():
                        prefetch(k + 1, 1 - p)
                    consume(k, p)
            return carry

        lax.fori_loop(0, (k1 - k0 + 1) // 2, pair_body, 0)

        @pl.when(k0 < k1)
        def _():
            last_p = (k1 - 1 - k0) & 1
            @pl.when(last_p == 0)
            def _():
                wait_scatter(0)

            @pl.when(last_p == 1)
            def _():
                wait_scatter(1)

        pltpu.sync_copy(acc.at[pl.ds(s * SEG_PT, SEG_PT)],
                        out_ref.at[c].at[pl.ds(seg_base + s * SEG_PT,
                                               SEG_PT)])

        @pl.when(last)
        def _():
            pltpu.sync_copy(acc.at[pl.ds(NS * SEG_PT, tail)],
                            out_ref.at[c].at[pl.ds(seg_base + NS * SEG_PT,
                                                   tail)])
        plsc.subcore_barrier()

    def run_side(tab_x, tab_y, xlist, ylist, ptr_hbm, nrow_x, nrow_y,
                 out_ref):
        pltpu.sync_copy(ptr_hbm, ptrbuf)

        def pass_body(i, carry):
            run_pass(tab_x, tab_y, xlist, ylist, i * NSEG_HALF,
                     nrow_x, nrow_y, out_ref)
            return carry

        lax.fori_loop(0, NT // NSEG_HALF, pass_body, 0)

    run_side(tab_vt, tab_at, vlist_t, alist_t, ptr_t, NV, NA, out_t)
    run_side(tab_tv, tab_av, tlist_v, alist_v, ptr_v, NT, NA, out_v)


_sc_segsum = functools.partial(
    pl.kernel,
    out_type=[jax.ShapeDtypeStruct((NC, NT, H), jnp.float32),
              jax.ShapeDtypeStruct((NC, NV, H), jnp.float32)],
    mesh=plsc.VectorSubcoreMesh(core_axis_name="c", subcore_axis_name="s"),
    compiler_params=pltpu.CompilerParams(needs_layout_passes=False),
    scratch_types=[
        pltpu.VMEM((PTR_LEN,), jnp.int32),
        pltpu.VMEM((CH,), jnp.int32),
        pltpu.VMEM((CH,), jnp.int32),
        pltpu.VMEM((CH,), jnp.int32),
        pltpu.VMEM((CH, H), jnp.float32),
        pltpu.VMEM((CH, H), jnp.float32),
        pltpu.VMEM((CH,), jnp.int32),
        pltpu.VMEM((CH,), jnp.int32),
        pltpu.VMEM((CH,), jnp.int32),
        pltpu.VMEM((CH, H), jnp.float32),
        pltpu.VMEM((CH, H), jnp.float32),
        pltpu.VMEM((ZROWS, H), jnp.float32),
        pltpu.VMEM_SHARED((R_SP, H), jnp.float32),
        pltpu.SemaphoreType.DMA,
        pltpu.SemaphoreType.DMA,
        pltpu.SemaphoreType.DMA,
        pltpu.SemaphoreType.DMA,
        pltpu.SemaphoreType.DMA,
    ],
)(_sc_body)


def kernel(t_embed, v_embed, a_embed, wv, wt, wa_t, w1, w2, wa, wa_v,
           ptr_t, a_list_t, v_list_t, ptr_v, a_list_v, t_list_v):
    f32 = jnp.float32
    i32 = jnp.int32
    t_embed = t_embed.astype(f32)
    v_embed = v_embed.astype(f32)
    a_embed = a_embed.astype(f32)

    tab_vt = _matmul_split(v_embed, wv.T)
    tab_at = _matmul_split(a_embed, wa_v.T)
    tab_tv = _matmul_split(t_embed, wt.T)
    tab_av = _matmul_split(a_embed, wa_t.T)

    vlt = v_list_t.astype(i32)
    alt = a_list_t.astype(i32)
    tlv = t_list_v.astype(i32)
    alv = a_list_v.astype(i32)
    ptrt = jnp.pad(ptr_t.astype(i32), (0, PTR_LEN - (NT + 1)), mode="edge")
    ptrv = jnp.pad(ptr_v.astype(i32), (0, PTR_LEN - (NV + 1)), mode="edge")

    out_t2, out_v2 = _sc_segsum(tab_vt, tab_at, vlt, alt, ptrt,
                                tab_tv, tab_av, tlv, alv, ptrv)
    seg_t = jnp.concatenate([out_t2[0], out_t2[1]], axis=1)
    seg_v = jnp.concatenate([out_v2[0], out_v2[1]], axis=1)

    t_up = _matmul2(t_embed, w1[:, :D].T, seg_t, w1[:, D:].T)
    v_up = _matmul2(v_embed, w2[:, :D].T, seg_v, w2[:, D:].T)
    a_out = _matmul(a_embed, wa)
    return (t_up, v_up, a_out)


# speedup vs baseline: 41.2979x; 1.2537x over previous
import functools

import jax
import jax.numpy as jnp
from jax import lax
from jax.experimental import pallas as pl
from jax.experimental.pallas import tpu as pltpu
from jax.experimental.pallas import tpu_sc as plsc

NT = 10000
NV = 10000
NA = 10000
E = 160000
D = 256
H = 128
NC = 2
NS = 16
L = 16
CH = 128
NCHUNK = E // CH
PTR_LEN = 10016
NSEG_HALF = 5000
SEG_PT = 312
R_SP = 5016
ZROWS = 96
BSEARCH_ITERS = 9

BN = 1000


def _mm_kernel(x_ref, w_ref, o_ref):
    o_ref[...] = jnp.dot(x_ref[...], w_ref[...],
                         preferred_element_type=jnp.float32)


def _matmul(x, w):
    n, k = x.shape
    m = w.shape[1]
    return pl.pallas_call(
        _mm_kernel,
        grid=(n // BN,),
        in_specs=[pl.BlockSpec((BN, k), lambda i: (i, 0)),
                  pl.BlockSpec((k, m), lambda i: (0, 0))],
        out_specs=pl.BlockSpec((BN, m), lambda i: (i, 0)),
        out_shape=jax.ShapeDtypeStruct((n, m), jnp.float32),
    )(x, w)


def _mm_split_kernel(x_ref, w_ref, o_ref):
    o_ref[0] = jnp.dot(x_ref[...], w_ref[...],
                       preferred_element_type=jnp.float32)


def _matmul_split(x, w):
    n = x.shape[0]
    out = pl.pallas_call(
        _mm_split_kernel,
        grid=(NC, n // BN),
        in_specs=[pl.BlockSpec((BN, D), lambda c, i: (i, 0)),
                  pl.BlockSpec((D, H), lambda c, i: (0, c))],
        out_specs=pl.BlockSpec((1, BN, H), lambda c, i: (c, i, 0)),
        out_shape=jax.ShapeDtypeStruct((NC, n, H), jnp.float32),
    )(x, w)
    return out.reshape(NC * n, H)


def _mm2_kernel(x1_ref, w1_ref, x2_ref, w2_ref, o_ref):
    o_ref[...] = (
        jnp.dot(x1_ref[...], w1_ref[...], preferred_element_type=jnp.float32)
        + jnp.dot(x2_ref[...], w2_ref[...],
                  preferred_element_type=jnp.float32))


def _matmul2(x1, w1_, x2, w2_):
    n, k1 = x1.shape
    k2 = x2.shape[1]
    m = w1_.shape[1]
    return pl.pallas_call(
        _mm2_kernel,
        grid=(n // BN,),
        in_specs=[pl.BlockSpec((BN, k1), lambda i: (i, 0)),
                  pl.BlockSpec((k1, m), lambda i: (0, 0)),
                  pl.BlockSpec((BN, k2), lambda i: (i, 0)),
                  pl.BlockSpec((k2, m), lambda i: (0, 0))],
        out_specs=pl.BlockSpec((BN, m), lambda i: (i, 0)),
        out_shape=jax.ShapeDtypeStruct((n, m), jnp.float32),
    )(x1, w1_, x2, w2_)


def _sc_body(tab_vt, tab_at, vlist_t, alist_t, ptr_t,
             tab_tv, tab_av, tlist_v, alist_v, ptr_v,
             out_t, out_v,
             ptrbuf, iXbuf, iYbuf, segbuf, xbuf, ybuf,
             iXbuf2, iYbuf2, segbuf2, xbuf2, ybuf2, zbuf, acc,
             semx0, semy0, semx1, semy1, semi0, semj0, semi1, semj1, sems):
    c = lax.axis_index("c")
    s = lax.axis_index("s")
    zeros = jnp.zeros((L,), jnp.float32)
    last = s == NS - 1

    def zrow(r, carry):
        for g in range(H // L):
            zbuf[r, pl.ds(g * L, L)] = zeros
        return carry
    lax.fori_loop(0, ZROWS, zrow, 0)

    def run_pass(tab_x, tab_y, xlist, ylist, seg_base, nrow_x, nrow_y,
                 out_ref):
        seg_lo = seg_base + s * SEG_PT
        seg_hi = seg_base + jnp.where(last, NSEG_HALF, (s + 1) * SEG_PT)
        trash = NSEG_HALF + s
        tail = NSEG_HALF - NS * SEG_PT

        for off in range(0, SEG_PT, ZROWS):
            step = min(ZROWS, SEG_PT - off)
            pltpu.sync_copy(zbuf.at[pl.ds(0, step)],
                            acc.at[pl.ds(s * SEG_PT + off, step)])

        @pl.when(last)
        def _():
            pltpu.sync_copy(zbuf.at[pl.ds(0, tail)],
                            acc.at[pl.ds(NS * SEG_PT, tail)])

        p_lo = plsc.load_gather(
            ptrbuf, [jnp.broadcast_to(seg_lo, (L,)).astype(jnp.int32)])
        p_hi = plsc.load_gather(
            ptrbuf, [jnp.broadcast_to(seg_hi, (L,)).astype(jnp.int32)])
        k0 = p_lo[0] // CH
        k1 = (p_hi[0] + (CH - 1)) // CH
        lovec = jnp.broadcast_to(seg_lo, (L,)).astype(jnp.int32)
        hivec = jnp.broadcast_to(seg_hi - 1, (L,)).astype(jnp.int32)
        off_x = c * nrow_x
        off_y = c * nrow_y

        bufs = ((iXbuf, iYbuf, segbuf, xbuf, ybuf, semx0, semy0,
                 semi0, semj0),
                (iXbuf2, iYbuf2, segbuf2, xbuf2, ybuf2, semx1, semy1,
                 semi1, semj1))

        def start_idx(k, bi):
            iXb, iYb, sgb, xb, yb, sx, sy, si, sj = bufs[bi]
            base = k * CH
            pltpu.async_copy(xlist.at[pl.ds(base, CH)], iXb, si)
            pltpu.async_copy(ylist.at[pl.ds(base, CH)], iYb, sj)

        def prefetch(k, bi):
            iXb, iYb, sgb, xb, yb, sx, sy, si, sj = bufs[bi]
            base = k * CH
            pltpu.make_async_copy(xlist.at[pl.ds(base, CH)], iXb, si).wait()
            pltpu.make_async_copy(ylist.at[pl.ds(base, CH)], iYb, sj).wait()
            for g in range(CH // L):
                sl = pl.ds(g * L, L)
                iXb[sl] = iXb[sl] + off_x
                iYb[sl] = iYb[sl] + off_y
                e = (jnp.full((L,), base + g * L, jnp.int32)
                     + lax.iota(jnp.int32, L))
                own = (e >= p_lo) & (e < p_hi)
                lo = lovec
                hi = hivec
                for _ in range(BSEARCH_ITERS):
                    mid = (lo + hi + 1) >> 1
                    pm = plsc.load_gather(ptrbuf, [mid])
                    gele = pm <= e
                    lo = jnp.where(gele, mid, lo)
                    hi = jnp.where(gele, hi, mid - 1)
                sgb[sl] = jnp.where(own, lo - seg_base, trash)
            pltpu.async_copy(tab_x.at[iXb], xb, sx)
            pltpu.async_copy(tab_y.at[iYb], yb, sy)

        def consume(k, bi):
            iXb, iYb, sgb, xb, yb, sx, sy, si, sj = bufs[bi]
            pltpu.make_async_copy(tab_x.at[iXb], xb, sx).wait()
            pltpu.make_async_copy(tab_y.at[iYb], yb, sy).wait()

            @pl.when(k + 2 < k1)
            def _():
                start_idx(k + 2, bi)

            def mrow(r, mcarry):
                for g in range(H // L):
                    sl2 = pl.ds(g * L, L)
                    xb[r, sl2] = xb[r, sl2] * yb[r, sl2]
                return mcarry
            lax.fori_loop(0, CH, mrow, 0)
            pltpu.async_copy(xb, acc.at[sgb], sems, add=True)

        def wait_scatter(bi):
            iXb, iYb, sgb, xb, yb, sx, sy, si, sj = bufs[bi]
            pltpu.make_async_copy(xb, acc.at[sgb], sems).wait()

        @pl.when(k0 < k1)
        def _():
            start_idx(k0, 0)

            @pl.when(k0 + 1 < k1)
            def _():
                start_idx(k0 + 1, 1)
            prefetch(k0, 0)

        def pair_body(j, carry):
            for p in range(2):
                k = k0 + 2 * j + p

                @pl.when(k < k1)
                def _():
                    @pl.when(k > k0)
                    def _():
                        wait_scatter(1 - p)

                    @pl.when(k + 1 < k1)
                    def _():
                        prefetch(k + 1, 1 - p)
                    consume(k, p)
            return carry

        lax.fori_loop(0, (k1 - k0 + 1) // 2, pair_body, 0)

        @pl.when(k0 < k1)
        def _():
            last_p = (k1 - 1 - k0) & 1
            @pl.when(last_p == 0)
            def _():
                wait_scatter(0)

            @pl.when(last_p == 1)
            def _():
                wait_scatter(1)

        pltpu.sync_copy(acc.at[pl.ds(s * SEG_PT, SEG_PT)],
                        out_ref.at[c].at[pl.ds(seg_base + s * SEG_PT,
                                               SEG_PT)])

        @pl.when(last)
        def _():
            pltpu.sync_copy(acc.at[pl.ds(NS * SEG_PT, tail)],
                            out_ref.at[c].at[pl.ds(seg_base + NS * SEG_PT,
                                                   tail)])
        plsc.subcore_barrier()

    def run_side(tab_x, tab_y, xlist, ylist, ptr_hbm, nrow_x, nrow_y,
                 out_ref):
        pltpu.sync_copy(ptr_hbm, ptrbuf)

        def pass_body(i, carry):
            run_pass(tab_x, tab_y, xlist, ylist, i * NSEG_HALF,
                     nrow_x, nrow_y, out_ref)
            return carry

        lax.fori_loop(0, NT // NSEG_HALF, pass_body, 0)

    run_side(tab_vt, tab_at, vlist_t, alist_t, ptr_t, NV, NA, out_t)
    run_side(tab_tv, tab_av, tlist_v, alist_v, ptr_v, NT, NA, out_v)


_sc_segsum = functools.partial(
    pl.kernel,
    out_type=[jax.ShapeDtypeStruct((NC, NT, H), jnp.float32),
              jax.ShapeDtypeStruct((NC, NV, H), jnp.float32)],
    mesh=plsc.VectorSubcoreMesh(core_axis_name="c", subcore_axis_name="s"),
    compiler_params=pltpu.CompilerParams(needs_layout_passes=False),
    scratch_types=[
        pltpu.VMEM((PTR_LEN,), jnp.int32),
        pltpu.VMEM((CH,), jnp.int32),
        pltpu.VMEM((CH,), jnp.int32),
        pltpu.VMEM((CH,), jnp.int32),
        pltpu.VMEM((CH, H), jnp.float32),
        pltpu.VMEM((CH, H), jnp.float32),
        pltpu.VMEM((CH,), jnp.int32),
        pltpu.VMEM((CH,), jnp.int32),
        pltpu.VMEM((CH,), jnp.int32),
        pltpu.VMEM((CH, H), jnp.float32),
        pltpu.VMEM((CH, H), jnp.float32),
        pltpu.VMEM((ZROWS, H), jnp.float32),
        pltpu.VMEM_SHARED((R_SP, H), jnp.float32),
        pltpu.SemaphoreType.DMA,
        pltpu.SemaphoreType.DMA,
        pltpu.SemaphoreType.DMA,
        pltpu.SemaphoreType.DMA,
        pltpu.SemaphoreType.DMA,
        pltpu.SemaphoreType.DMA,
        pltpu.SemaphoreType.DMA,
        pltpu.SemaphoreType.DMA,
        pltpu.SemaphoreType.DMA,
    ],
)(_sc_body)


def kernel(t_embed, v_embed, a_embed, wv, wt, wa_t, w1, w2, wa, wa_v,
           ptr_t, a_list_t, v_list_t, ptr_v, a_list_v, t_list_v):
    f32 = jnp.float32
    i32 = jnp.int32
    t_embed = t_embed.astype(f32)
    v_embed = v_embed.astype(f32)
    a_embed = a_embed.astype(f32)

    tab_vt = _matmul_split(v_embed, wv.T)
    tab_at = _matmul_split(a_embed, wa_v.T)
    tab_tv = _matmul_split(t_embed, wt.T)
    tab_av = _matmul_split(a_embed, wa_t.T)

    vlt = v_list_t.astype(i32)
    alt = a_list_t.astype(i32)
    tlv = t_list_v.astype(i32)
    alv = a_list_v.astype(i32)
    ptrt = jnp.pad(ptr_t.astype(i32), (0, PTR_LEN - (NT + 1)), mode="edge")
    ptrv = jnp.pad(ptr_v.astype(i32), (0, PTR_LEN - (NV + 1)), mode="edge")

    out_t2, out_v2 = _sc_segsum(tab_vt, tab_at, vlt, alt, ptrt,
                                tab_tv, tab_av, tlv, alv, ptrv)
    seg_t = jnp.concatenate([out_t2[0], out_t2[1]], axis=1)
    seg_v = jnp.concatenate([out_v2[0], out_v2[1]], axis=1)

    t_up = _matmul2(t_embed, w1[:, :D].T, seg_t, w1[:, D:].T)
    v_up = _matmul2(v_embed, w2[:, :D].T, seg_v, w2[:, D:].T)
    a_out = _matmul(a_embed, wa)
    return (t_up, v_up, a_out)


# speedup vs baseline: 41.3484x; 1.0012x over previous
import functools

import jax
import jax.numpy as jnp
from jax import lax
from jax.experimental import pallas as pl
from jax.experimental.pallas import tpu as pltpu
from jax.experimental.pallas import tpu_sc as plsc

NT = 10000
NV = 10000
NA = 10000
E = 160000
D = 256
H = 128
NC = 2
NS = 16
L = 16
CH = 128
NCHUNK = E // CH
PTR_LEN = 10016
NSEG_HALF = 5000
SEG_PT = 312
R_SP = 5016
ZROWS = 96
BSEARCH_ITERS = 9

BN = 1000


def _mm_kernel(x_ref, w_ref, o_ref):
    o_ref[...] = jnp.dot(x_ref[...], w_ref[...],
                         preferred_element_type=jnp.float32)


def _matmul(x, w):
    n, k = x.shape
    m = w.shape[1]
    return pl.pallas_call(
        _mm_kernel,
        grid=(n // BN,),
        in_specs=[pl.BlockSpec((BN, k), lambda i: (i, 0)),
                  pl.BlockSpec((k, m), lambda i: (0, 0))],
        out_specs=pl.BlockSpec((BN, m), lambda i: (i, 0)),
        out_shape=jax.ShapeDtypeStruct((n, m), jnp.float32),
    )(x, w)


def _mm_split_kernel(x_ref, w_ref, o_ref):
    o_ref[0] = jnp.dot(x_ref[...], w_ref[...],
                       preferred_element_type=jnp.float32)


def _matmul_split(x, w):
    n = x.shape[0]
    out = pl.pallas_call(
        _mm_split_kernel,
        grid=(NC, n // BN),
        in_specs=[pl.BlockSpec((BN, D), lambda c, i: (i, 0)),
                  pl.BlockSpec((D, H), lambda c, i: (0, c))],
        out_specs=pl.BlockSpec((1, BN, H), lambda c, i: (c, i, 0)),
        out_shape=jax.ShapeDtypeStruct((NC, n, H), jnp.float32),
    )(x, w)
    return out.reshape(NC * n, H)


def _mm2_kernel(x1_ref, w1_ref, x2_ref, w2_ref, o_ref):
    o_ref[...] = (
        jnp.dot(x1_ref[...], w1_ref[...], preferred_element_type=jnp.float32)
        + jnp.dot(x2_ref[...], w2_ref[...],
                  preferred_element_type=jnp.float32))


def _matmul2(x1, w1_, x2, w2_):
    n, k1 = x1.shape
    k2 = x2.shape[1]
    m = w1_.shape[1]
    return pl.pallas_call(
        _mm2_kernel,
        grid=(n // BN,),
        in_specs=[pl.BlockSpec((BN, k1), lambda i: (i, 0)),
                  pl.BlockSpec((k1, m), lambda i: (0, 0)),
                  pl.BlockSpec((BN, k2), lambda i: (i, 0)),
                  pl.BlockSpec((k2, m), lambda i: (0, 0))],
        out_specs=pl.BlockSpec((BN, m), lambda i: (i, 0)),
        out_shape=jax.ShapeDtypeStruct((n, m), jnp.float32),
    )(x1, w1_, x2, w2_)


def _sc_body(tab_vt, tab_at, vlist_t, alist_t, ptr_t,
             tab_tv, tab_av, tlist_v, alist_v, ptr_v,
             out_t, out_v,
             ptrbuf, iXbuf, iYbuf, segbuf, xbuf, ybuf,
             iXbuf2, iYbuf2, segbuf2, xbuf2, ybuf2, zbuf, acc,
             semx0, semy0, semx1, semy1, semi0, semj0, semi1, semj1, sems):
    c = lax.axis_index("c")
    s = lax.axis_index("s")
    zeros = jnp.zeros((L,), jnp.float32)
    last = s == NS - 1

    def zrow(r, carry):
        for g in range(H // L):
            zbuf[r, pl.ds(g * L, L)] = zeros
        return carry
    lax.fori_loop(0, ZROWS, zrow, 0)

    def run_pass(tab_x, tab_y, xlist, ylist, seg_base, nrow_x, nrow_y,
                 out_ref):
        seg_lo = seg_base + s * SEG_PT
        seg_hi = seg_base + jnp.where(last, NSEG_HALF, (s + 1) * SEG_PT)
        trash = NSEG_HALF + s
        tail = NSEG_HALF - NS * SEG_PT

        for off in range(0, SEG_PT, ZROWS):
            step = min(ZROWS, SEG_PT - off)
            pltpu.sync_copy(zbuf.at[pl.ds(0, step)],
                            acc.at[pl.ds(s * SEG_PT + off, step)])

        @pl.when(last)
        def _():
            pltpu.sync_copy(zbuf.at[pl.ds(0, tail)],
                            acc.at[pl.ds(NS * SEG_PT, tail)])

        p_lo = plsc.load_gather(
            ptrbuf, [jnp.broadcast_to(seg_lo, (L,)).astype(jnp.int32)])
        p_hi = plsc.load_gather(
            ptrbuf, [jnp.broadcast_to(seg_hi, (L,)).astype(jnp.int32)])
        k0 = p_lo[0] // CH
        k1 = (p_hi[0] + (CH - 1)) // CH
        lovec = jnp.broadcast_to(seg_lo, (L,)).astype(jnp.int32)
        hivec = jnp.broadcast_to(seg_hi - 1, (L,)).astype(jnp.int32)
        off_x = c * nrow_x
        off_y = c * nrow_y

        bufs = ((iXbuf, iYbuf, segbuf, xbuf, ybuf, semx0, semy0,
                 semi0, semj0),
                (iXbuf2, iYbuf2, segbuf2, xbuf2, ybuf2, semx1, semy1,
                 semi1, semj1))

        def start_idx(k, bi):
            iXb, iYb, sgb, xb, yb, sx, sy, si, sj = bufs[bi]
            base = k * CH
            pltpu.async_copy(xlist.at[pl.ds(base, CH)], iXb, si)
            pltpu.async_copy(ylist.at[pl.ds(base, CH)], iYb, sj)

        def prefetch(k, bi):
            iXb, iYb, sgb, xb, yb, sx, sy, si, sj = bufs[bi]
            base = k * CH
            pltpu.make_async_copy(xlist.at[pl.ds(base, CH)], iXb, si).wait()
            pltpu.make_async_copy(ylist.at[pl.ds(base, CH)], iYb, sj).wait()
            for g in range(CH // L):
                sl = pl.ds(g * L, L)
                iXb[sl] = iXb[sl] + off_x
                iYb[sl] = iYb[sl] + off_y
                e = (jnp.full((L,), base + g * L, jnp.int32)
                     + lax.iota(jnp.int32, L))
                own = (e >= p_lo) & (e < p_hi)
                lo = lovec
                hi = hivec
                for _ in range(BSEARCH_ITERS):
                    mid = (lo + hi + 1) >> 1
                    pm = plsc.load_gather(ptrbuf, [mid])
                    gele = pm <= e
                    lo = jnp.where(gele, mid, lo)
                    hi = jnp.where(gele, hi, mid - 1)
                sgb[sl] = jnp.where(own, lo - seg_base, trash)
            pltpu.async_copy(tab_x.at[iXb], xb, sx)
            pltpu.async_copy(tab_y.at[iYb], yb, sy)

        def consume(k, bi):
            iXb, iYb, sgb, xb, yb, sx, sy, si, sj = bufs[bi]
            pltpu.make_async_copy(tab_x.at[iXb], xb, sx).wait()
            pltpu.make_async_copy(tab_y.at[iYb], yb, sy).wait()

            @pl.when(k + 2 < k1)
            def _():
                start_idx(k + 2, bi)

            def mrow(r, mcarry):
                for rr in range(4):
                    row = 4 * r + rr
                    for g in range(H // L):
                        sl2 = pl.ds(g * L, L)
                        xb[row, sl2] = xb[row, sl2] * yb[row, sl2]
                return mcarry
            lax.fori_loop(0, CH // 4, mrow, 0)
            pltpu.async_copy(xb, acc.at[sgb], sems, add=True)

        def wait_scatter(bi):
            iXb, iYb, sgb, xb, yb, sx, sy, si, sj = bufs[bi]
            pltpu.make_async_copy(xb, acc.at[sgb], sems).wait()

        @pl.when(k0 < k1)
        def _():
            start_idx(k0, 0)

            @pl.when(k0 + 1 < k1)
            def _():
                start_idx(k0 + 1, 1)
            prefetch(k0, 0)

        def pair_body(j, carry):
            for p in range(2):
                k = k0 + 2 * j + p

                @pl.when(k < k1)
                def _():
                    @pl.when(k > k0)
                    def _():
                        wait_scatter(1 - p)

                    @pl.when(k + 1 < k1)
                    def _():
                        prefetch(k + 1, 1 - p)
                    consume(k, p)
            return carry

        lax.fori_loop(0, (k1 - k0 + 1) // 2, pair_body, 0)

        @pl.when(k0 < k1)
        def _():
            last_p = (k1 - 1 - k0) & 1
            @pl.when(last_p == 0)
            def _():
                wait_scatter(0)

            @pl.when(last_p == 1)
            def _():
                wait_scatter(1)

        pltpu.sync_copy(acc.at[pl.ds(s * SEG_PT, SEG_PT)],
                        out_ref.at[c].at[pl.ds(seg_base + s * SEG_PT,
                                               SEG_PT)])

        @pl.when(last)
        def _():
            pltpu.sync_copy(acc.at[pl.ds(NS * SEG_PT, tail)],
                            out_ref.at[c].at[pl.ds(seg_base + NS * SEG_PT,
                                                   tail)])
        plsc.subcore_barrier()

    def run_side(tab_x, tab_y, xlist, ylist, ptr_hbm, nrow_x, nrow_y,
                 out_ref):
        pltpu.sync_copy(ptr_hbm, ptrbuf)

        def pass_body(i, carry):
            run_pass(tab_x, tab_y, xlist, ylist, i * NSEG_HALF,
                     nrow_x, nrow_y, out_ref)
            return carry

        lax.fori_loop(0, NT // NSEG_HALF, pass_body, 0)

    run_side(tab_vt, tab_at, vlist_t, alist_t, ptr_t, NV, NA, out_t)
    run_side(tab_tv, tab_av, tlist_v, alist_v, ptr_v, NT, NA, out_v)


_sc_segsum = functools.partial(
    pl.kernel,
    out_type=[jax.ShapeDtypeStruct((NC, NT, H), jnp.float32),
              jax.ShapeDtypeStruct((NC, NV, H), jnp.float32)],
    mesh=plsc.VectorSubcoreMesh(core_axis_name="c", subcore_axis_name="s"),
    compiler_params=pltpu.CompilerParams(needs_layout_passes=False),
    scratch_types=[
        pltpu.VMEM((PTR_LEN,), jnp.int32),
        pltpu.VMEM((CH,), jnp.int32),
        pltpu.VMEM((CH,), jnp.int32),
        pltpu.VMEM((CH,), jnp.int32),
        pltpu.VMEM((CH, H), jnp.float32),
        pltpu.VMEM((CH, H), jnp.float32),
        pltpu.VMEM((CH,), jnp.int32),
        pltpu.VMEM((CH,), jnp.int32),
        pltpu.VMEM((CH,), jnp.int32),
        pltpu.VMEM((CH, H), jnp.float32),
        pltpu.VMEM((CH, H), jnp.float32),
        pltpu.VMEM((ZROWS, H), jnp.float32),
        pltpu.VMEM_SHARED((R_SP, H), jnp.float32),
        pltpu.SemaphoreType.DMA,
        pltpu.SemaphoreType.DMA,
        pltpu.SemaphoreType.DMA,
        pltpu.SemaphoreType.DMA,
        pltpu.SemaphoreType.DMA,
        pltpu.SemaphoreType.DMA,
        pltpu.SemaphoreType.DMA,
        pltpu.SemaphoreType.DMA,
        pltpu.SemaphoreType.DMA,
    ],
)(_sc_body)


def kernel(t_embed, v_embed, a_embed, wv, wt, wa_t, w1, w2, wa, wa_v,
           ptr_t, a_list_t, v_list_t, ptr_v, a_list_v, t_list_v):
    f32 = jnp.float32
    i32 = jnp.int32
    t_embed = t_embed.astype(f32)
    v_embed = v_embed.astype(f32)
    a_embed = a_embed.astype(f32)

    tab_vt = _matmul_split(v_embed, wv.T)
    tab_at = _matmul_split(a_embed, wa_v.T)
    tab_tv = _matmul_split(t_embed, wt.T)
    tab_av = _matmul_split(a_embed, wa_t.T)

    vlt = v_list_t.astype(i32)
    alt = a_list_t.astype(i32)
    tlv = t_list_v.astype(i32)
    alv = a_list_v.astype(i32)
    ptrt = jnp.pad(ptr_t.astype(i32), (0, PTR_LEN - (NT + 1)), mode="edge")
    ptrv = jnp.pad(ptr_v.astype(i32), (0, PTR_LEN - (NV + 1)), mode="edge")

    out_t2, out_v2 = _sc_segsum(tab_vt, tab_at, vlt, alt, ptrt,
                                tab_tv, tab_av, tlv, alv, ptrv)
    seg_t = jnp.concatenate([out_t2[0], out_t2[1]], axis=1)
    seg_v = jnp.concatenate([out_v2[0], out_v2[1]], axis=1)

    t_up = _matmul2(t_embed, w1[:, :D].T, seg_t, w1[:, D:].T)
    v_up = _matmul2(v_embed, w2[:, :D].T, seg_v, w2[:, D:].T)
    a_out = _matmul(a_embed, wa)
    return (t_up, v_up, a_out)
